# baseline (speedup 1.0000x reference)
import jax
import jax.numpy as jnp
from jax import lax
from jax.experimental import pallas as pl
from jax.experimental.pallas import tpu as pltpu
from jax.experimental.pallas import tpu_sc as plsc

B = 4096
L = 50
ROWS = 2 * B
WD = 32
ED = 64
PD = 2
NW = 32
RPW = ROWS // NW
IPW = RPW * L
CHUNK = 128
NCHUNK = IPW // CHUNK
WBLK = 8000
EBLK = 5000


def _proj_body(tab_ref, w0_ref, w1_ref, out_ref):
    blk = tab_ref[...]
    c0 = jnp.sum(blk * w0_ref[...], axis=1, keepdims=True)
    c1 = jnp.sum(blk * w1_ref[...], axis=1, keepdims=True)
    out_ref[...] = jnp.concatenate([c0, c1], axis=1)


def _project(table, w0, w1, blk):
    v, d = table.shape
    return pl.pallas_call(
        _proj_body,
        grid=(v // blk,),
        in_specs=[
            pl.BlockSpec((blk, d), lambda i: (i, 0)),
            pl.BlockSpec((1, d), lambda i: (0, 0)),
            pl.BlockSpec((1, d), lambda i: (0, 0)),
        ],
        out_specs=pl.BlockSpec((blk, PD), lambda i: (i, 0)),
        out_shape=jax.ShapeDtypeStruct((v, PD), jnp.float32),
    )(table, w0, w1)


def _pool_body(wp_hbm, ep_hbm, widx_hbm, eidx_hbm, gidx_hbm, z_hbm,
               outw_hbm, oute_hbm,
               idxw_v, idxe_v, g_v, buf0, buf1, accw_s, acce_s,
               sem0, sem1):
    c = lax.axis_index("c")
    s = lax.axis_index("s")
    w = c * 16 + s

    pltpu.sync_copy(widx_hbm.at[w], idxw_v)
    pltpu.sync_copy(eidx_hbm.at[w], idxe_v)
    pltpu.sync_copy(gidx_hbm.at[s], g_v)

    pltpu.sync_copy(z_hbm, accw_s.at[pl.ds(s * RPW, RPW)])
    pltpu.sync_copy(z_hbm, acce_s.at[pl.ds(s * RPW, RPW)])

    def phase(tab_hbm, idx_v, acc_s):
        pltpu.async_copy(tab_hbm.at[idx_v.at[0]], buf0, sem0).wait()

        def step(j, carry):
            i = 2 * j
            pltpu.async_copy(tab_hbm.at[idx_v.at[i + 1]], buf1, sem1)
            pltpu.sync_copy(buf0, acc_s.at[g_v.at[i]], add=True)
            pltpu.make_async_copy(tab_hbm.at[idx_v.at[i + 1]], buf1,
                                  sem1).wait()

            @pl.when(i + 2 < NCHUNK)
            def _():
                pltpu.async_copy(tab_hbm.at[idx_v.at[i + 2]], buf0, sem0)

            pltpu.sync_copy(buf1, acc_s.at[g_v.at[i + 1]], add=True)

            @pl.when(i + 2 < NCHUNK)
            def _():
                pltpu.make_async_copy(tab_hbm.at[idx_v.at[i + 2]], buf0,
                                      sem0).wait()

            return carry

        lax.fori_loop(0, NCHUNK // 2, step, 0, unroll=False)

    phase(wp_hbm, idxw_v, accw_s)
    phase(ep_hbm, idxe_v, acce_s)

    pltpu.sync_copy(accw_s.at[pl.ds(s * RPW, RPW)],
                    outw_hbm.at[pl.ds(w * RPW, RPW)])
    pltpu.sync_copy(acce_s.at[pl.ds(s * RPW, RPW)],
                    oute_hbm.at[pl.ds(w * RPW, RPW)])


def _scorer_body(params_ref, wsum_ref, esum_ref, out_ref):
    wsum = wsum_ref[...]
    esum = esum_ref[...]
    er_raw = wsum[:, 0:1]
    el_raw = esum[:, 0:1]
    ec_raw = wsum[:, 1:2] + esum[:, 1:2]
    er_b, el_b, ec_b, cl_b = (params_ref[0], params_ref[1], params_ref[2],
                              params_ref[3])
    cl0, cl1, cl2 = params_ref[4], params_ref[5], params_ref[6]
    er = jax.nn.relu(er_raw + er_b - 0.5) + 0.5
    el = jax.nn.relu(el_raw + el_b - 0.5) + 0.5
    ec = jax.nn.sigmoid(ec_raw + ec_b)
    out_ref[...] = jax.nn.sigmoid(er * cl0 + el * cl1 + ec * cl2 + cl_b)


def kernel(lctx_words, rctx_words, lctx_entities, rctx_entities,
           word_table, entity_table, er_w, er_b, el_w, el_b,
           ec_w, ec_b, cl_w, cl_b):
    widx = jnp.concatenate([rctx_words, lctx_words], axis=0).reshape(
        NW, NCHUNK, CHUNK)
    eidx = jnp.concatenate([rctx_entities, lctx_entities], axis=0).reshape(
        NW, NCHUNK, CHUNK)
    seg = (jnp.arange(IPW, dtype=jnp.int32) // L).reshape(1, NCHUNK, CHUNK)
    gidx = seg + (RPW * jnp.arange(16, dtype=jnp.int32))[:, None, None]
    z = jnp.zeros((RPW, PD), jnp.float32)

    inv_l = 1.0 / L
    wproj = _project(word_table, er_w.reshape(1, WD) * inv_l,
                     ec_w[:WD].reshape(1, WD) * inv_l, WBLK)
    eproj = _project(entity_table, el_w.reshape(1, ED) * inv_l,
                     ec_w[WD:].reshape(1, ED) * inv_l, EBLK)

    mesh = plsc.VectorSubcoreMesh(core_axis_name="c", subcore_axis_name="s")
    pool = pl.kernel(
        _pool_body,
        out_type=(jax.ShapeDtypeStruct((ROWS, PD), jnp.float32),
                  jax.ShapeDtypeStruct((ROWS, PD), jnp.float32)),
        mesh=mesh,
        compiler_params=pltpu.CompilerParams(use_tc_tiling_on_sc=False),
        scratch_types=[
            pltpu.VMEM((NCHUNK, CHUNK), jnp.int32),
            pltpu.VMEM((NCHUNK, CHUNK), jnp.int32),
            pltpu.VMEM((NCHUNK, CHUNK), jnp.int32),
            pltpu.VMEM((CHUNK, PD), jnp.float32),
            pltpu.VMEM((CHUNK, PD), jnp.float32),
            pltpu.VMEM_SHARED((16 * RPW, PD), jnp.float32),
            pltpu.VMEM_SHARED((16 * RPW, PD), jnp.float32),
            pltpu.SemaphoreType.DMA,
            pltpu.SemaphoreType.DMA,
        ],
    )
    wsum, esum = pool(wproj, eproj, widx, eidx, gidx, z)

    params = jnp.concatenate([er_b, el_b, ec_b, cl_b, cl_w[:, 0]])
    final = pl.pallas_call(
        _scorer_body,
        out_shape=jax.ShapeDtypeStruct((ROWS, 1), jnp.float32),
        in_specs=[
            pl.BlockSpec(memory_space=pltpu.SMEM),
            pl.BlockSpec(memory_space=pltpu.VMEM),
            pl.BlockSpec(memory_space=pltpu.VMEM),
        ],
    )(params, wsum, esum)
    return final


# speedup vs baseline: 1.6031x; 1.6031x over previous
import jax
import jax.numpy as jnp
from jax import lax
from jax.experimental import pallas as pl
from jax.experimental.pallas import tpu as pltpu
from jax.experimental.pallas import tpu_sc as plsc

B = 4096
L = 50
ROWS = 2 * B
WD = 32
ED = 64
WV = 1000000
EV = 100000
NW = 32
RPW = ROWS // NW
IPW = RPW * L
CHUNK = 128
NCHUNK = IPW // CHUNK
NCHUNK_PAD = 104
PBLK = 2048


def _proj_body(tab_ref, m_ref, *o_refs):
    m = m_ref[...].astype(jnp.bfloat16)
    out = jax.lax.dot_general(m, tab_ref[...].astype(jnp.bfloat16),
                              (((1,), (1,)), ((), ())),
                              preferred_element_type=jnp.float32)
    for c, o_ref in enumerate(o_refs):
        o_ref[...] = out[c]


def _project(table128, m, ncols):
    v128, _ = table128.shape
    return pl.pallas_call(
        _proj_body,
        grid=(pl.cdiv(v128, PBLK),),
        in_specs=[
            pl.BlockSpec((PBLK, 128), lambda i: (i, 0)),
            pl.BlockSpec((128, 128), lambda i: (0, 0)),
        ],
        out_specs=[pl.BlockSpec((PBLK,), lambda i: (i,))] * ncols,
        out_shape=[jax.ShapeDtypeStruct((v128,), jnp.float32)] * ncols,
    )(table128, m)


def _pool_body(wcat_hbm, ecat_hbm, w0i_hbm, w1i_hbm, e0i_hbm, e1i_hbm,
               gidx_hbm, z_hbm, ow0_hbm, ow1_hbm, oe0_hbm, oe1_hbm,
               idx_v, g_v, buf0, buf1,
               aw0_s, aw1_s, ae0_s, ae1_s, sem0, sem1):
    c = lax.axis_index("c")
    s = lax.axis_index("s")
    w = c * 16 + s

    pltpu.sync_copy(gidx_hbm.at[s], g_v)

    for acc in (aw0_s, aw1_s, ae0_s, ae1_s):
        pltpu.sync_copy(z_hbm, acc.at[pl.ds(s * RPW, RPW)])

    def phase(col_hbm, idxsrc_hbm, acc_s):
        pltpu.sync_copy(idxsrc_hbm.at[w], idx_v)
        pltpu.async_copy(col_hbm.at[idx_v.at[pl.ds(0, CHUNK)]], buf0,
                         sem0).wait()

        def step(j, carry):
            i = 2 * j

            def idx(n):
                return idx_v.at[pl.ds(n * CHUNK, CHUNK)]

            pltpu.async_copy(col_hbm.at[idx(i + 1)], buf1, sem1)
            pltpu.sync_copy(buf0, acc_s.at[g_v.at[i]], add=True)
            pltpu.make_async_copy(col_hbm.at[idx(i + 1)], buf1, sem1).wait()

            @pl.when(i + 2 < NCHUNK)
            def _():
                pltpu.async_copy(col_hbm.at[idx(i + 2)], buf0, sem0)

            pltpu.sync_copy(buf1, acc_s.at[g_v.at[i + 1]], add=True)

            @pl.when(i + 2 < NCHUNK)
            def _():
                pltpu.make_async_copy(col_hbm.at[idx(i + 2)], buf0,
                                      sem0).wait()

            return carry

        lax.fori_loop(0, NCHUNK // 2, step, 0, unroll=False)

    phase(wcat_hbm, w0i_hbm, aw0_s)
    phase(wcat_hbm, w1i_hbm, aw1_s)
    phase(ecat_hbm, e0i_hbm, ae0_s)
    phase(ecat_hbm, e1i_hbm, ae1_s)

    for acc, out in ((aw0_s, ow0_hbm), (aw1_s, ow1_hbm),
                     (ae0_s, oe0_hbm), (ae1_s, oe1_hbm)):
        pltpu.sync_copy(acc.at[pl.ds(s * RPW, RPW)],
                        out.at[pl.ds(w * RPW, RPW)])


def _scorer_body(params_ref, er_ref, ecw_ref, el_ref, ece_ref, out_ref):
    er_raw = er_ref[...]
    el_raw = el_ref[...]
    ec_raw = ecw_ref[...] + ece_ref[...]
    er_b, el_b, ec_b, cl_b = (params_ref[0], params_ref[1], params_ref[2],
                              params_ref[3])
    cl0, cl1, cl2 = params_ref[4], params_ref[5], params_ref[6]
    er = jax.nn.relu(er_raw + er_b - 0.5) + 0.5
    el = jax.nn.relu(el_raw + el_b - 0.5) + 0.5
    ec = jax.nn.sigmoid(ec_raw + ec_b)
    out_ref[...] = jax.nn.sigmoid(er * cl0 + el * cl1 + ec * cl2 + cl_b)


def _projmat(w0, w1, d, packs):
    m = jnp.zeros((128, 128), jnp.float32)
    for q in range(packs):
        m = m.at[2 * q, q * d:(q + 1) * d].set(w0)
        m = m.at[2 * q + 1, q * d:(q + 1) * d].set(w1)
    return m


def kernel(lctx_words, rctx_words, lctx_entities, rctx_entities,
           word_table, entity_table, er_w, er_b, el_w, el_b,
           ec_w, ec_b, cl_w, cl_b):
    inv_l = 1.0 / L
    wcols = _project(word_table.reshape(-1, 128),
                     _projmat(er_w[:, 0] * inv_l, ec_w[:WD, 0] * inv_l,
                              WD, 128 // WD), 2 * (128 // WD))
    ecols = _project(entity_table.reshape(-1, 128),
                     _projmat(el_w[:, 0] * inv_l, ec_w[WD:, 0] * inv_l,
                              ED, 128 // ED), 2 * (128 // ED))
    wcat = jnp.concatenate([wcols[2 * q + k] for k in range(2)
                            for q in range(128 // WD)])
    ecat = jnp.concatenate([ecols[2 * q + k] for k in range(2)
                            for q in range(128 // ED)])

    widx = jnp.concatenate([rctx_words, lctx_words], axis=0).reshape(
        NW, IPW)
    eidx = jnp.concatenate([rctx_entities, lctx_entities], axis=0).reshape(
        NW, IPW)
    w0i = (widx & 3) * (WV // 4) + (widx >> 2)
    w1i = w0i + WV
    e0i = (eidx & 1) * (EV // 2) + (eidx >> 1)
    e1i = e0i + EV

    seg = (jnp.arange(IPW, dtype=jnp.int32) // L).reshape(NCHUNK, CHUNK)
    seg = jnp.pad(seg, ((0, NCHUNK_PAD - NCHUNK), (0, 0)))[None]
    gidx = seg + (RPW * jnp.arange(16, dtype=jnp.int32))[:, None, None]
    z = jnp.zeros((RPW,), jnp.float32)

    mesh = plsc.VectorSubcoreMesh(core_axis_name="c", subcore_axis_name="s")
    pool = pl.kernel(
        _pool_body,
        out_type=tuple(jax.ShapeDtypeStruct((ROWS,), jnp.float32)
                       for _ in range(4)),
        mesh=mesh,
        compiler_params=pltpu.CompilerParams(use_tc_tiling_on_sc=False),
        scratch_types=[
            pltpu.VMEM((IPW,), jnp.int32),
            pltpu.VMEM((NCHUNK_PAD, CHUNK), jnp.int32),
            pltpu.VMEM((CHUNK,), jnp.float32),
            pltpu.VMEM((CHUNK,), jnp.float32),
            pltpu.VMEM_SHARED((16 * RPW,), jnp.float32),
            pltpu.VMEM_SHARED((16 * RPW,), jnp.float32),
            pltpu.VMEM_SHARED((16 * RPW,), jnp.float32),
            pltpu.VMEM_SHARED((16 * RPW,), jnp.float32),
            pltpu.SemaphoreType.DMA,
            pltpu.SemaphoreType.DMA,
        ],
    )
    ow0, ow1, oe0, oe1 = pool(wcat, ecat, w0i, w1i, e0i, e1i, gidx, z)

    params = jnp.concatenate([er_b, el_b, ec_b, cl_b, cl_w[:, 0]])
    final = pl.pallas_call(
        _scorer_body,
        out_shape=jax.ShapeDtypeStruct((ROWS,), jnp.float32),
        in_specs=[pl.BlockSpec(memory_space=pltpu.SMEM)] +
                 [pl.BlockSpec(memory_space=pltpu.VMEM)] * 4,
    )(params, ow0, ow1, oe0, oe1)
    return final.reshape(ROWS, 1)


# speedup vs baseline: 1.9942x; 1.2439x over previous
import jax
import jax.numpy as jnp
from jax import lax
from jax.experimental import pallas as pl
from jax.experimental.pallas import tpu as pltpu
from jax.experimental.pallas import tpu_sc as plsc

B = 4096
L = 50
ROWS = 2 * B
WD = 32
ED = 64
WV = 1000000
EV = 100000
NW = 32
RPW = ROWS // NW
IPW = RPW * L
CHUNK = 128
NCHUNK = IPW // CHUNK
NCHUNK_PAD = 104
SUP = 10
NSUP = NCHUNK // SUP
PBLK = 2048


def _proj_body(tab_ref, m_ref, *o_refs):
    m = m_ref[...].astype(jnp.bfloat16)
    out = jax.lax.dot_general(m, tab_ref[...].astype(jnp.bfloat16),
                              (((1,), (1,)), ((), ())),
                              preferred_element_type=jnp.float32)
    for c, o_ref in enumerate(o_refs):
        o_ref[...] = out[c]


def _project(table128, m, ncols):
    v128, _ = table128.shape
    return pl.pallas_call(
        _proj_body,
        grid=(pl.cdiv(v128, PBLK),),
        in_specs=[
            pl.BlockSpec((PBLK, 128), lambda i: (i, 0)),
            pl.BlockSpec((128, 128), lambda i: (0, 0)),
        ],
        out_specs=[pl.BlockSpec((PBLK,), lambda i: (i,))] * ncols,
        out_shape=[jax.ShapeDtypeStruct((v128,), jnp.float32)] * ncols,
    )(table128, m)


def _pool_body(wcat_hbm, ecat_hbm, w0i_hbm, w1i_hbm, e0i_hbm, e1i_hbm,
               gidx_hbm, z_hbm, ow0_hbm, ow1_hbm, oe0_hbm, oe1_hbm,
               idx_v, g_v, buf0, buf1,
               aw0_s, aw1_s, ae0_s, ae1_s, sem0, sem1, ssem):
    c = lax.axis_index("c")
    s = lax.axis_index("s")
    w = c * 16 + s

    pltpu.sync_copy(gidx_hbm.at[s], g_v)

    for acc in (aw0_s, aw1_s, ae0_s, ae1_s):
        pltpu.sync_copy(z_hbm, acc.at[pl.ds(s * RPW, RPW)])

    def phase(col_hbm, idxsrc_hbm, acc_s):
        pltpu.sync_copy(idxsrc_hbm.at[w], idx_v)

        def fire(i, buf, sem):
            ds = []
            for t in range(SUP):
                src = col_hbm.at[idx_v.at[pl.ds((i * SUP + t) * CHUNK,
                                                CHUNK)]]
                ds.append(pltpu.async_copy(src, buf.at[pl.ds(t * CHUNK,
                                                             CHUNK)], sem))
            return ds

        def drain(i, buf, sem):
            for t in range(SUP):
                src = col_hbm.at[idx_v.at[pl.ds((i * SUP + t) * CHUNK,
                                                CHUNK)]]
                pltpu.make_async_copy(src, buf.at[pl.ds(t * CHUNK, CHUNK)],
                                      sem).wait()

        def scat(i, buf):
            ds = []
            for t in range(SUP):
                ds.append(pltpu.async_copy(
                    buf.at[pl.ds(t * CHUNK, CHUNK)],
                    acc_s.at[g_v.at[i * SUP + t]], ssem, add=True))
            for d in ds:
                d.wait()

        fire(0, buf0, sem0)

        def step(j, carry):
            i = 2 * j
            drain(i, buf0, sem0)
            fire(i + 1, buf1, sem1)
            scat(i, buf0)
            drain(i + 1, buf1, sem1)

            @pl.when(i + 2 < NSUP)
            def _():
                fire(i + 2, buf0, sem0)

            scat(i + 1, buf1)
            return carry

        lax.fori_loop(0, NSUP // 2, step, 0, unroll=False)

    phase(wcat_hbm, w0i_hbm, aw0_s)
    phase(wcat_hbm, w1i_hbm, aw1_s)
    phase(ecat_hbm, e0i_hbm, ae0_s)
    phase(ecat_hbm, e1i_hbm, ae1_s)

    for acc, out in ((aw0_s, ow0_hbm), (aw1_s, ow1_hbm),
                     (ae0_s, oe0_hbm), (ae1_s, oe1_hbm)):
        pltpu.sync_copy(acc.at[pl.ds(s * RPW, RPW)],
                        out.at[pl.ds(w * RPW, RPW)])


def _scorer_body(params_ref, er_ref, ecw_ref, el_ref, ece_ref, out_ref):
    er_raw = er_ref[...]
    el_raw = el_ref[...]
    ec_raw = ecw_ref[...] + ece_ref[...]
    er_b, el_b, ec_b, cl_b = (params_ref[0], params_ref[1], params_ref[2],
                              params_ref[3])
    cl0, cl1, cl2 = params_ref[4], params_ref[5], params_ref[6]
    er = jax.nn.relu(er_raw + er_b - 0.5) + 0.5
    el = jax.nn.relu(el_raw + el_b - 0.5) + 0.5
    ec = jax.nn.sigmoid(ec_raw + ec_b)
    out_ref[...] = jax.nn.sigmoid(er * cl0 + el * cl1 + ec * cl2 + cl_b)


def _projmat(w0, w1, d, packs):
    m = jnp.zeros((128, 128), jnp.float32)
    for q in range(packs):
        m = m.at[2 * q, q * d:(q + 1) * d].set(w0)
        m = m.at[2 * q + 1, q * d:(q + 1) * d].set(w1)
    return m


def kernel(lctx_words, rctx_words, lctx_entities, rctx_entities,
           word_table, entity_table, er_w, er_b, el_w, el_b,
           ec_w, ec_b, cl_w, cl_b):
    inv_l = 1.0 / L
    wcols = _project(word_table.reshape(-1, 128),
                     _projmat(er_w[:, 0] * inv_l, ec_w[:WD, 0] * inv_l,
                              WD, 128 // WD), 2 * (128 // WD))
    ecols = _project(entity_table.reshape(-1, 128),
                     _projmat(el_w[:, 0] * inv_l, ec_w[WD:, 0] * inv_l,
                              ED, 128 // ED), 2 * (128 // ED))
    wcat = jnp.concatenate([wcols[2 * q + k] for k in range(2)
                            for q in range(128 // WD)])
    ecat = jnp.concatenate([ecols[2 * q + k] for k in range(2)
                            for q in range(128 // ED)])

    widx = jnp.concatenate([rctx_words, lctx_words], axis=0).reshape(
        NW, IPW)
    eidx = jnp.concatenate([rctx_entities, lctx_entities], axis=0).reshape(
        NW, IPW)
    w0i = (widx & 3) * (WV // 4) + (widx >> 2)
    w1i = w0i + WV
    e0i = (eidx & 1) * (EV // 2) + (eidx >> 1)
    e1i = e0i + EV

    seg = (jnp.arange(IPW, dtype=jnp.int32) // L).reshape(NCHUNK, CHUNK)
    seg = jnp.pad(seg, ((0, NCHUNK_PAD - NCHUNK), (0, 0)))[None]
    gidx = seg + (RPW * jnp.arange(16, dtype=jnp.int32))[:, None, None]
    z = jnp.zeros((RPW,), jnp.float32)

    mesh = plsc.VectorSubcoreMesh(core_axis_name="c", subcore_axis_name="s")
    pool = pl.kernel(
        _pool_body,
        out_type=tuple(jax.ShapeDtypeStruct((ROWS,), jnp.float32)
                       for _ in range(4)),
        mesh=mesh,
        compiler_params=pltpu.CompilerParams(use_tc_tiling_on_sc=False),
        scratch_types=[
            pltpu.VMEM((IPW,), jnp.int32),
            pltpu.VMEM((NCHUNK_PAD, CHUNK), jnp.int32),
            pltpu.VMEM((SUP * CHUNK,), jnp.float32),
            pltpu.VMEM((SUP * CHUNK,), jnp.float32),
            pltpu.VMEM_SHARED((16 * RPW,), jnp.float32),
            pltpu.VMEM_SHARED((16 * RPW,), jnp.float32),
            pltpu.VMEM_SHARED((16 * RPW,), jnp.float32),
            pltpu.VMEM_SHARED((16 * RPW,), jnp.float32),
            pltpu.SemaphoreType.DMA,
            pltpu.SemaphoreType.DMA,
            pltpu.SemaphoreType.DMA,
        ],
    )
    ow0, ow1, oe0, oe1 = pool(wcat, ecat, w0i, w1i, e0i, e1i, gidx, z)

    params = jnp.concatenate([er_b, el_b, ec_b, cl_b, cl_w[:, 0]])
    final = pl.pallas_call(
        _scorer_body,
        out_shape=jax.ShapeDtypeStruct((ROWS,), jnp.float32),
        in_specs=[pl.BlockSpec(memory_space=pltpu.SMEM)] +
                 [pl.BlockSpec(memory_space=pltpu.VMEM)] * 4,
    )(params, ow0, ow1, oe0, oe1)
    return final.reshape(ROWS, 1)


# speedup vs baseline: 2.1767x; 1.0915x over previous
import jax
import jax.numpy as jnp
from jax import lax
from jax.experimental import pallas as pl
from jax.experimental.pallas import tpu as pltpu
from jax.experimental.pallas import tpu_sc as plsc

B = 4096
L = 50
ROWS = 2 * B
WD = 32
ED = 64
WV = 1000000
EV = 100000
NW = 32
RPW = ROWS // NW
IPW = RPW * L
CHUNK = 128
NCHUNK = IPW // CHUNK
NCHUNK_PAD = 104
SUP = 10
NSUP = NCHUNK // SUP
PBLK = 2048


def _proj_body(tab_ref, m_ref, *o_refs):
    m = m_ref[...].astype(jnp.bfloat16)
    out = jax.lax.dot_general(m, tab_ref[...].astype(jnp.bfloat16),
                              (((1,), (1,)), ((), ())),
                              preferred_element_type=jnp.float32)

    def bf_hi(x):
        r = x.astype(jnp.bfloat16).astype(jnp.float32)
        return jax.lax.bitcast_convert_type(r, jnp.uint32) & jnp.uint32(
            0xFFFF0000)

    for q, o_ref in enumerate(o_refs):
        packed = bf_hi(out[2 * q]) | (bf_hi(out[2 * q + 1]) >> 16)
        o_ref[...] = jax.lax.bitcast_convert_type(packed, jnp.int32)


def _project(table128, m, ncols):
    v128, _ = table128.shape
    return pl.pallas_call(
        _proj_body,
        grid=(pl.cdiv(v128, PBLK),),
        in_specs=[
            pl.BlockSpec((PBLK, 128), lambda i: (i, 0)),
            pl.BlockSpec((128, 128), lambda i: (0, 0)),
        ],
        out_specs=[pl.BlockSpec((PBLK,), lambda i: (i,))] * ncols,
        out_shape=[jax.ShapeDtypeStruct((v128,), jnp.int32)] * ncols,
    )(table128, m)


def _pool_body(wcat_hbm, ecat_hbm, wpi_hbm, epi_hbm,
               gidx_hbm, z_hbm, ow0_hbm, ow1_hbm, oe0_hbm, oe1_hbm,
               idx_v, g_v, buf0, buf1, abuf, bbuf,
               aw0_s, aw1_s, ae0_s, ae1_s, sem0, sem1, ssem):
    c = lax.axis_index("c")
    s = lax.axis_index("s")
    w = c * 16 + s

    pltpu.sync_copy(gidx_hbm.at[s], g_v)

    for acc in (aw0_s, aw1_s, ae0_s, ae1_s):
        pltpu.sync_copy(z_hbm, acc.at[pl.ds(s * RPW, RPW)])

    def phase(col_hbm, idxsrc_hbm, acc0_s, acc1_s):
        pltpu.sync_copy(idxsrc_hbm.at[w], idx_v)

        def fire(i, buf, sem):
            for t in range(SUP):
                src = col_hbm.at[idx_v.at[pl.ds((i * SUP + t) * CHUNK,
                                                CHUNK)]]
                pltpu.async_copy(src, buf.at[pl.ds(t * CHUNK, CHUNK)], sem)

        def drain(i, buf, sem):
            for t in range(SUP):
                src = col_hbm.at[idx_v.at[pl.ds((i * SUP + t) * CHUNK,
                                                CHUNK)]]
                pltpu.make_async_copy(src, buf.at[pl.ds(t * CHUNK, CHUNK)],
                                      sem).wait()

        def unpack_scat(i, buf):
            for r in range(SUP * CHUNK // 16):
                v = buf[pl.ds(r * 16, 16)]
                abuf[pl.ds(r * 16, 16)] = plsc.bitcast(
                    v & jnp.int32(-65536), jnp.float32)
                bbuf[pl.ds(r * 16, 16)] = plsc.bitcast(
                    jax.lax.shift_left(v, 16), jnp.float32)
            ds = []
            for t in range(SUP):
                g = g_v.at[i * SUP + t]
                ds.append(pltpu.async_copy(
                    abuf.at[pl.ds(t * CHUNK, CHUNK)],
                    acc0_s.at[g], ssem, add=True))
                ds.append(pltpu.async_copy(
                    bbuf.at[pl.ds(t * CHUNK, CHUNK)],
                    acc1_s.at[g], ssem, add=True))
            for d in ds:
                d.wait()

        fire(0, buf0, sem0)

        def step(j, carry):
            i = 2 * j
            drain(i, buf0, sem0)
            fire(i + 1, buf1, sem1)
            unpack_scat(i, buf0)
            drain(i + 1, buf1, sem1)

            @pl.when(i + 2 < NSUP)
            def _():
                fire(i + 2, buf0, sem0)

            unpack_scat(i + 1, buf1)
            return carry

        lax.fori_loop(0, NSUP // 2, step, 0, unroll=False)

    phase(wcat_hbm, wpi_hbm, aw0_s, aw1_s)
    phase(ecat_hbm, epi_hbm, ae0_s, ae1_s)

    for acc, out in ((aw0_s, ow0_hbm), (aw1_s, ow1_hbm),
                     (ae0_s, oe0_hbm), (ae1_s, oe1_hbm)):
        pltpu.sync_copy(acc.at[pl.ds(s * RPW, RPW)],
                        out.at[pl.ds(w * RPW, RPW)])


def _scorer_body(params_ref, er_ref, ecw_ref, el_ref, ece_ref, out_ref):
    er_raw = er_ref[...]
    el_raw = el_ref[...]
    ec_raw = ecw_ref[...] + ece_ref[...]
    er_b, el_b, ec_b, cl_b = (params_ref[0], params_ref[1], params_ref[2],
                              params_ref[3])
    cl0, cl1, cl2 = params_ref[4], params_ref[5], params_ref[6]
    er = jax.nn.relu(er_raw + er_b - 0.5) + 0.5
    el = jax.nn.relu(el_raw + el_b - 0.5) + 0.5
    ec = jax.nn.sigmoid(ec_raw + ec_b)
    out_ref[...] = jax.nn.sigmoid(er * cl0 + el * cl1 + ec * cl2 + cl_b)


def _projmat(w0, w1, d, packs):
    m = jnp.zeros((128, 128), jnp.float32)
    for q in range(packs):
        m = m.at[2 * q, q * d:(q + 1) * d].set(w0)
        m = m.at[2 * q + 1, q * d:(q + 1) * d].set(w1)
    return m


def kernel(lctx_words, rctx_words, lctx_entities, rctx_entities,
           word_table, entity_table, er_w, er_b, el_w, el_b,
           ec_w, ec_b, cl_w, cl_b):
    inv_l = 1.0 / L
    wcols = _project(word_table.reshape(-1, 128),
                     _projmat(er_w[:, 0] * inv_l, ec_w[:WD, 0] * inv_l,
                              WD, 128 // WD), 128 // WD)
    ecols = _project(entity_table.reshape(-1, 128),
                     _projmat(el_w[:, 0] * inv_l, ec_w[WD:, 0] * inv_l,
                              ED, 128 // ED), 128 // ED)
    wcat = jnp.concatenate(wcols)
    ecat = jnp.concatenate(ecols)

    widx = jnp.concatenate([rctx_words, lctx_words], axis=0).reshape(
        NW, IPW)
    eidx = jnp.concatenate([rctx_entities, lctx_entities], axis=0).reshape(
        NW, IPW)
    wpi = (widx & 3) * (WV // 4) + (widx >> 2)
    epi = (eidx & 1) * (EV // 2) + (eidx >> 1)

    seg = (jnp.arange(IPW, dtype=jnp.int32) // L).reshape(NCHUNK, CHUNK)
    seg = jnp.pad(seg, ((0, NCHUNK_PAD - NCHUNK), (0, 0)))[None]
    gidx = seg + (RPW * jnp.arange(16, dtype=jnp.int32))[:, None, None]
    z = jnp.zeros((RPW,), jnp.float32)

    mesh = plsc.VectorSubcoreMesh(core_axis_name="c", subcore_axis_name="s")
    pool = pl.kernel(
        _pool_body,
        out_type=tuple(jax.ShapeDtypeStruct((ROWS,), jnp.float32)
                       for _ in range(4)),
        mesh=mesh,
        compiler_params=pltpu.CompilerParams(use_tc_tiling_on_sc=False,
                                             needs_layout_passes=False),
        scratch_types=[
            pltpu.VMEM((IPW,), jnp.int32),
            pltpu.VMEM((NCHUNK_PAD, CHUNK), jnp.int32),
            pltpu.VMEM((SUP * CHUNK,), jnp.int32),
            pltpu.VMEM((SUP * CHUNK,), jnp.int32),
            pltpu.VMEM((SUP * CHUNK,), jnp.float32),
            pltpu.VMEM((SUP * CHUNK,), jnp.float32),
            pltpu.VMEM_SHARED((16 * RPW,), jnp.float32),
            pltpu.VMEM_SHARED((16 * RPW,), jnp.float32),
            pltpu.VMEM_SHARED((16 * RPW,), jnp.float32),
            pltpu.VMEM_SHARED((16 * RPW,), jnp.float32),
            pltpu.SemaphoreType.DMA,
            pltpu.SemaphoreType.DMA,
            pltpu.SemaphoreType.DMA,
        ],
    )
    ow0, ow1, oe0, oe1 = pool(wcat, ecat, wpi, epi, gidx, z)

    params = jnp.concatenate([er_b, el_b, ec_b, cl_b, cl_w[:, 0]])
    final = pl.pallas_call(
        _scorer_body,
        out_shape=jax.ShapeDtypeStruct((ROWS,), jnp.float32),
        in_specs=[pl.BlockSpec(memory_space=pltpu.SMEM)] +
                 [pl.BlockSpec(memory_space=pltpu.VMEM)] * 4,
    )(params, ow0, ow1, oe0, oe1)
    return final.reshape(ROWS, 1)


# speedup vs baseline: 2.2431x; 1.0305x over previous
import jax
import jax.numpy as jnp
from jax import lax
from jax.experimental import pallas as pl
from jax.experimental.pallas import tpu as pltpu
from jax.experimental.pallas import tpu_sc as plsc

B = 4096
L = 50
ROWS = 2 * B
WD = 32
ED = 64
WV = 1000000
EV = 100000
NW = 32
RPW = ROWS // NW
IPW = RPW * L
CHUNK = 128
NCHUNK = IPW // CHUNK
NCHUNK_PAD = 104
SUP = 10
NSUP = NCHUNK // SUP
PBLK = 2048


def _proj_body(tab_ref, m_ref, *o_refs):
    m = m_ref[...].astype(jnp.bfloat16)
    out = jax.lax.dot_general(m, tab_ref[...].astype(jnp.bfloat16),
                              (((1,), (1,)), ((), ())),
                              preferred_element_type=jnp.float32)

    def bf_hi(x):
        r = x.astype(jnp.bfloat16).astype(jnp.float32)
        return jax.lax.bitcast_convert_type(r, jnp.uint32) & jnp.uint32(
            0xFFFF0000)

    for q, o_ref in enumerate(o_refs):
        packed = bf_hi(out[2 * q]) | (bf_hi(out[2 * q + 1]) >> 16)
        o_ref[...] = jax.lax.bitcast_convert_type(packed, jnp.int32)


def _project(table128, m, ncols):
    v128, _ = table128.shape
    return pl.pallas_call(
        _proj_body,
        grid=(pl.cdiv(v128, PBLK),),
        in_specs=[
            pl.BlockSpec((PBLK, 128), lambda i: (i, 0)),
            pl.BlockSpec((128, 128), lambda i: (0, 0)),
        ],
        out_specs=[pl.BlockSpec((PBLK,), lambda i: (i,))] * ncols,
        out_shape=[jax.ShapeDtypeStruct((v128,), jnp.int32)] * ncols,
    )(table128, m)


def _pool_body(wcat_hbm, ecat_hbm, wpi_hbm, epi_hbm,
               gidx_hbm, z_hbm, ow0_hbm, ow1_hbm, oe0_hbm, oe1_hbm,
               idx_v, g_v, buf0, buf1, abuf, bbuf,
               aw0_s, aw1_s, ae0_s, ae1_s, sem0, sem1, ssem):
    c = lax.axis_index("c")
    s = lax.axis_index("s")
    w = c * 16 + s

    pltpu.sync_copy(gidx_hbm.at[s], g_v)

    for acc in (aw0_s, aw1_s, ae0_s, ae1_s):
        pltpu.sync_copy(z_hbm, acc.at[pl.ds(s * RPW, RPW)])

    def phase(col_hbm, idxsrc_hbm, acc0_s, acc1_s):
        pltpu.sync_copy(idxsrc_hbm.at[w], idx_v)

        def fire(i, buf, sem):
            for t in range(SUP):
                src = col_hbm.at[idx_v.at[pl.ds((i * SUP + t) * CHUNK,
                                                CHUNK)]]
                pltpu.async_copy(src, buf.at[pl.ds(t * CHUNK, CHUNK)], sem)

        def drain(i, buf, sem):
            for t in range(SUP):
                src = col_hbm.at[idx_v.at[pl.ds((i * SUP + t) * CHUNK,
                                                CHUNK)]]
                pltpu.make_async_copy(src, buf.at[pl.ds(t * CHUNK, CHUNK)],
                                      sem).wait()

        def unpack_scat(i, buf):
            for r in range(SUP * CHUNK // 16):
                v = buf[pl.ds(r * 16, 16)]
                abuf[pl.ds(r * 16, 16)] = plsc.bitcast(
                    v & jnp.int32(-65536), jnp.float32)
                bbuf[pl.ds(r * 16, 16)] = plsc.bitcast(
                    jax.lax.shift_left(v, 16), jnp.float32)
            ds = []
            for t in range(SUP):
                g = g_v.at[i * SUP + t]
                ds.append(pltpu.async_copy(
                    abuf.at[pl.ds(t * CHUNK, CHUNK)],
                    acc0_s.at[g], ssem, add=True))
                ds.append(pltpu.async_copy(
                    bbuf.at[pl.ds(t * CHUNK, CHUNK)],
                    acc1_s.at[g], ssem, add=True))
            for d in ds:
                d.wait()

        fire(0, buf0, sem0)

        def step(j, carry):
            i = 2 * j
            drain(i, buf0, sem0)
            fire(i + 1, buf1, sem1)
            unpack_scat(i, buf0)
            drain(i + 1, buf1, sem1)

            @pl.when(i + 2 < NSUP)
            def _():
                fire(i + 2, buf0, sem0)

            unpack_scat(i + 1, buf1)
            return carry

        lax.fori_loop(0, NSUP // 2, step, 0, unroll=False)

    phase(wcat_hbm, wpi_hbm, aw0_s, aw1_s)
    phase(ecat_hbm, epi_hbm, ae0_s, ae1_s)

    for acc, out in ((aw0_s, ow0_hbm), (aw1_s, ow1_hbm),
                     (ae0_s, oe0_hbm), (ae1_s, oe1_hbm)):
        pltpu.sync_copy(acc.at[pl.ds(s * RPW, RPW)],
                        out.at[pl.ds(w * RPW, RPW)])


def _scorer_body(params_ref, er_ref, ecw_ref, el_ref, ece_ref, out_ref):
    er_raw = er_ref[...]
    el_raw = el_ref[...]
    ec_raw = ecw_ref[...] + ece_ref[...]
    er_b, el_b, ec_b, cl_b = (params_ref[0], params_ref[1], params_ref[2],
                              params_ref[3])
    cl0, cl1, cl2 = params_ref[4], params_ref[5], params_ref[6]
    er = jax.nn.relu(er_raw + er_b - 0.5) + 0.5
    el = jax.nn.relu(el_raw + el_b - 0.5) + 0.5
    ec = jax.nn.sigmoid(ec_raw + ec_b)
    out_ref[...] = jax.nn.sigmoid(er * cl0 + el * cl1 + ec * cl2 + cl_b)


def _projmat(w0, w1, d, packs):
    m = jnp.zeros((128, 128), jnp.float32)
    for q in range(packs):
        m = m.at[2 * q, q * d:(q + 1) * d].set(w0)
        m = m.at[2 * q + 1, q * d:(q + 1) * d].set(w1)
    return m


def kernel(lctx_words, rctx_words, lctx_entities, rctx_entities,
           word_table, entity_table, er_w, er_b, el_w, el_b,
           ec_w, ec_b, cl_w, cl_b):
    inv_l = 1.0 / L
    wcols = _project(word_table.reshape(-1, 128),
                     _projmat(er_w[:, 0] * inv_l, ec_w[:WD, 0] * inv_l,
                              WD, 128 // WD), 128 // WD)
    ecols = _project(entity_table.reshape(-1, 128),
                     _projmat(el_w[:, 0] * inv_l, ec_w[WD:, 0] * inv_l,
                              ED, 128 // ED), 128 // ED)
    wcat = jnp.concatenate(wcols)
    ecat = jnp.concatenate(ecols)

    widx = jnp.concatenate([rctx_words, lctx_words], axis=0).reshape(
        NW, IPW)
    eidx = jnp.concatenate([rctx_entities, lctx_entities], axis=0).reshape(
        NW, IPW)
    wpi = (widx & 3) * (WV // 4) + (widx >> 2)
    epi = (eidx & 1) * (EV // 2) + (eidx >> 1)

    def perm(a):
        return a.reshape(NW, CHUNK, NCHUNK).swapaxes(1, 2).reshape(NW, IPW)

    wpi, epi = perm(wpi), perm(epi)

    seg = (jnp.arange(IPW, dtype=jnp.int32) // L).reshape(1, IPW)
    seg = perm(jnp.broadcast_to(seg, (NW, IPW)))[0]
    seg = seg.reshape(NCHUNK, CHUNK)
    seg = jnp.pad(seg, ((0, NCHUNK_PAD - NCHUNK), (0, 0)))[None]
    gidx = seg + (RPW * jnp.arange(16, dtype=jnp.int32))[:, None, None]
    z = jnp.zeros((RPW,), jnp.float32)

    mesh = plsc.VectorSubcoreMesh(core_axis_name="c", subcore_axis_name="s")
    pool = pl.kernel(
        _pool_body,
        out_type=tuple(jax.ShapeDtypeStruct((ROWS,), jnp.float32)
                       for _ in range(4)),
        mesh=mesh,
        compiler_params=pltpu.CompilerParams(use_tc_tiling_on_sc=False,
                                             needs_layout_passes=False),
        scratch_types=[
            pltpu.VMEM((IPW,), jnp.int32),
            pltpu.VMEM((NCHUNK_PAD, CHUNK), jnp.int32),
            pltpu.VMEM((SUP * CHUNK,), jnp.int32),
            pltpu.VMEM((SUP * CHUNK,), jnp.int32),
            pltpu.VMEM((SUP * CHUNK,), jnp.float32),
            pltpu.VMEM((SUP * CHUNK,), jnp.float32),
            pltpu.VMEM_SHARED((16 * RPW,), jnp.float32),
            pltpu.VMEM_SHARED((16 * RPW,), jnp.float32),
            pltpu.VMEM_SHARED((16 * RPW,), jnp.float32),
            pltpu.VMEM_SHARED((16 * RPW,), jnp.float32),
            pltpu.SemaphoreType.DMA,
            pltpu.SemaphoreType.DMA,
            pltpu.SemaphoreType.DMA,
        ],
    )
    ow0, ow1, oe0, oe1 = pool(wcat, ecat, wpi, epi, gidx, z)

    params = jnp.concatenate([er_b, el_b, ec_b, cl_b, cl_w[:, 0]])
    final = pl.pallas_call(
        _scorer_body,
        out_shape=jax.ShapeDtypeStruct((ROWS,), jnp.float32),
        in_specs=[pl.BlockSpec(memory_space=pltpu.SMEM)] +
                 [pl.BlockSpec(memory_space=pltpu.VMEM)] * 4,
    )(params, ow0, ow1, oe0, oe1)
    return final.reshape(ROWS, 1)


# speedup vs baseline: 2.2853x; 1.0188x over previous
import jax
import jax.numpy as jnp
from jax import lax
from jax.experimental import pallas as pl
from jax.experimental.pallas import tpu as pltpu
from jax.experimental.pallas import tpu_sc as plsc

B = 4096
L = 50
ROWS = 2 * B
WD = 32
ED = 64
WV = 1000000
EV = 100000
NW = 32
RPW = ROWS // NW
IPW = RPW * L
CHUNK = 128
NCHUNK = IPW // CHUNK
NCHUNK_PAD = 104
SUP = 10
NSUP = NCHUNK // SUP
PBLK = 2048


def _proj_body(tab_ref, m_ref, *o_refs):
    m = m_ref[...].astype(jnp.bfloat16)
    out = jax.lax.dot_general(m, tab_ref[...].astype(jnp.bfloat16),
                              (((1,), (1,)), ((), ())),
                              preferred_element_type=jnp.float32)

    def bf_hi(x):
        r = x.astype(jnp.bfloat16).astype(jnp.float32)
        return jax.lax.bitcast_convert_type(r, jnp.uint32) & jnp.uint32(
            0xFFFF0000)

    for q, o_ref in enumerate(o_refs):
        packed = bf_hi(out[2 * q]) | (bf_hi(out[2 * q + 1]) >> 16)
        o_ref[...] = jax.lax.bitcast_convert_type(packed, jnp.int32)


def _project(table128, m, ncols):
    v128, _ = table128.shape
    return pl.pallas_call(
        _proj_body,
        grid=(pl.cdiv(v128, PBLK),),
        in_specs=[
            pl.BlockSpec((PBLK, 128), lambda i: (i, 0)),
            pl.BlockSpec((128, 128), lambda i: (0, 0)),
        ],
        out_specs=[pl.BlockSpec((PBLK,), lambda i: (i,))] * ncols,
        out_shape=[jax.ShapeDtypeStruct((v128,), jnp.int32)] * ncols,
    )(table128, m)


def _pool_body(wcat_hbm, ecat_hbm, wpi_hbm, epi_hbm,
               gidx_hbm, z_hbm, ow0_hbm, ow1_hbm, oe0_hbm, oe1_hbm,
               idx_v, g_v, buf0, buf1, abuf, bbuf, wtab_s, etab_s,
               aw0_s, aw1_s, ae0_s, ae1_s, sem0, sem1, ssem):
    c = lax.axis_index("c")
    s = lax.axis_index("s")
    w = c * 16 + s

    pltpu.sync_copy(gidx_hbm.at[s], g_v)

    @pl.when(s < 8)
    def _():
        pltpu.sync_copy(wcat_hbm.at[pl.ds(s * (WV // 8), WV // 8)],
                        wtab_s.at[pl.ds(s * (WV // 8), WV // 8)])

    @pl.when(s < 4)
    def _():
        pltpu.sync_copy(ecat_hbm.at[pl.ds(s * (EV // 4), EV // 4)],
                        etab_s.at[pl.ds(s * (EV // 4), EV // 4)])

    for acc in (aw0_s, aw1_s, ae0_s, ae1_s):
        pltpu.sync_copy(z_hbm, acc.at[pl.ds(s * RPW, RPW)])
    plsc.subcore_barrier()

    def phase(col_hbm, idxsrc_hbm, acc0_s, acc1_s):
        pltpu.sync_copy(idxsrc_hbm.at[w], idx_v)

        def fire(i, buf, sem):
            for t in range(SUP):
                src = col_hbm.at[idx_v.at[pl.ds((i * SUP + t) * CHUNK,
                                                CHUNK)]]
                pltpu.async_copy(src, buf.at[pl.ds(t * CHUNK, CHUNK)], sem)

        def drain(i, buf, sem):
            for t in range(SUP):
                src = col_hbm.at[idx_v.at[pl.ds((i * SUP + t) * CHUNK,
                                                CHUNK)]]
                pltpu.make_async_copy(src, buf.at[pl.ds(t * CHUNK, CHUNK)],
                                      sem).wait()

        def unpack_scat(i, buf):
            for r in range(SUP * CHUNK // 16):
                v = buf[pl.ds(r * 16, 16)]
                abuf[pl.ds(r * 16, 16)] = plsc.bitcast(
                    v & jnp.int32(-65536), jnp.float32)
                bbuf[pl.ds(r * 16, 16)] = plsc.bitcast(
                    jax.lax.shift_left(v, 16), jnp.float32)
            ds = []
            for t in range(SUP):
                g = g_v.at[i * SUP + t]
                ds.append(pltpu.async_copy(
                    abuf.at[pl.ds(t * CHUNK, CHUNK)],
                    acc0_s.at[g], ssem, add=True))
                ds.append(pltpu.async_copy(
                    bbuf.at[pl.ds(t * CHUNK, CHUNK)],
                    acc1_s.at[g], ssem, add=True))
            for d in ds:
                d.wait()

        fire(0, buf0, sem0)

        def step(j, carry):
            i = 2 * j
            drain(i, buf0, sem0)
            fire(i + 1, buf1, sem1)
            unpack_scat(i, buf0)
            drain(i + 1, buf1, sem1)

            @pl.when(i + 2 < NSUP)
            def _():
                fire(i + 2, buf0, sem0)

            unpack_scat(i + 1, buf1)
            return carry

        lax.fori_loop(0, NSUP // 2, step, 0, unroll=False)

    phase(wtab_s, wpi_hbm, aw0_s, aw1_s)
    phase(etab_s, epi_hbm, ae0_s, ae1_s)

    for acc, out in ((aw0_s, ow0_hbm), (aw1_s, ow1_hbm),
                     (ae0_s, oe0_hbm), (ae1_s, oe1_hbm)):
        pltpu.sync_copy(acc.at[pl.ds(s * RPW, RPW)],
                        out.at[pl.ds(w * RPW, RPW)])


def _scorer_body(params_ref, er_ref, ecw_ref, el_ref, ece_ref, out_ref):
    er_raw = er_ref[...]
    el_raw = el_ref[...]
    ec_raw = ecw_ref[...] + ece_ref[...]
    er_b, el_b, ec_b, cl_b = (params_ref[0], params_ref[1], params_ref[2],
                              params_ref[3])
    cl0, cl1, cl2 = params_ref[4], params_ref[5], params_ref[6]
    er = jax.nn.relu(er_raw + er_b - 0.5) + 0.5
    el = jax.nn.relu(el_raw + el_b - 0.5) + 0.5
    ec = jax.nn.sigmoid(ec_raw + ec_b)
    out_ref[...] = jax.nn.sigmoid(er * cl0 + el * cl1 + ec * cl2 + cl_b)


def _projmat(w0, w1, d, packs):
    m = jnp.zeros((128, 128), jnp.float32)
    for q in range(packs):
        m = m.at[2 * q, q * d:(q + 1) * d].set(w0)
        m = m.at[2 * q + 1, q * d:(q + 1) * d].set(w1)
    return m


def kernel(lctx_words, rctx_words, lctx_entities, rctx_entities,
           word_table, entity_table, er_w, er_b, el_w, el_b,
           ec_w, ec_b, cl_w, cl_b):
    inv_l = 1.0 / L
    wcols = _project(word_table.reshape(-1, 128),
                     _projmat(er_w[:, 0] * inv_l, ec_w[:WD, 0] * inv_l,
                              WD, 128 // WD), 128 // WD)
    ecols = _project(entity_table.reshape(-1, 128),
                     _projmat(el_w[:, 0] * inv_l, ec_w[WD:, 0] * inv_l,
                              ED, 128 // ED), 128 // ED)
    wcat = jnp.concatenate(wcols)
    ecat = jnp.concatenate(ecols)

    widx = jnp.concatenate([rctx_words, lctx_words], axis=0).reshape(
        NW, IPW)
    eidx = jnp.concatenate([rctx_entities, lctx_entities], axis=0).reshape(
        NW, IPW)
    wpi = (widx & 3) * (WV // 4) + (widx >> 2)
    epi = (eidx & 1) * (EV // 2) + (eidx >> 1)

    def perm(a):
        return a.reshape(NW, CHUNK, NCHUNK).swapaxes(1, 2).reshape(NW, IPW)

    wpi, epi = perm(wpi), perm(epi)

    seg = (jnp.arange(IPW, dtype=jnp.int32) // L).reshape(1, IPW)
    seg = perm(jnp.broadcast_to(seg, (NW, IPW)))[0]
    seg = seg.reshape(NCHUNK, CHUNK)
    seg = jnp.pad(seg, ((0, NCHUNK_PAD - NCHUNK), (0, 0)))[None]
    gidx = seg + (RPW * jnp.arange(16, dtype=jnp.int32))[:, None, None]
    z = jnp.zeros((RPW,), jnp.float32)

    mesh = plsc.VectorSubcoreMesh(core_axis_name="c", subcore_axis_name="s")
    pool = pl.kernel(
        _pool_body,
        out_type=tuple(jax.ShapeDtypeStruct((ROWS,), jnp.float32)
                       for _ in range(4)),
        mesh=mesh,
        compiler_params=pltpu.CompilerParams(use_tc_tiling_on_sc=False,
                                             needs_layout_passes=False),
        scratch_types=[
            pltpu.VMEM((IPW,), jnp.int32),
            pltpu.VMEM((NCHUNK_PAD, CHUNK), jnp.int32),
            pltpu.VMEM((SUP * CHUNK,), jnp.int32),
            pltpu.VMEM((SUP * CHUNK,), jnp.int32),
            pltpu.VMEM((SUP * CHUNK,), jnp.float32),
            pltpu.VMEM((SUP * CHUNK,), jnp.float32),
            pltpu.VMEM_SHARED((WV,), jnp.int32),
            pltpu.VMEM_SHARED((EV,), jnp.int32),
            pltpu.VMEM_SHARED((16 * RPW,), jnp.float32),
            pltpu.VMEM_SHARED((16 * RPW,), jnp.float32),
            pltpu.VMEM_SHARED((16 * RPW,), jnp.float32),
            pltpu.VMEM_SHARED((16 * RPW,), jnp.float32),
            pltpu.SemaphoreType.DMA,
            pltpu.SemaphoreType.DMA,
            pltpu.SemaphoreType.DMA,
        ],
    )
    ow0, ow1, oe0, oe1 = pool(wcat, ecat, wpi, epi, gidx, z)

    params = jnp.concatenate([er_b, el_b, ec_b, cl_b, cl_w[:, 0]])
    final = pl.pallas_call(
        _scorer_body,
        out_shape=jax.ShapeDtypeStruct((ROWS,), jnp.float32),
        in_specs=[pl.BlockSpec(memory_space=pltpu.SMEM)] +
                 [pl.BlockSpec(memory_space=pltpu.VMEM)] * 4,
    )(params, ow0, ow1, oe0, oe1)
    return final.reshape(ROWS, 1)


# speedup vs baseline: 2.2877x; 1.0010x over previous
import jax
import jax.numpy as jnp
from jax import lax
from jax.experimental import pallas as pl
from jax.experimental.pallas import tpu as pltpu
from jax.experimental.pallas import tpu_sc as plsc

B = 4096
L = 50
ROWS = 2 * B
WD = 32
ED = 64
WV = 1000000
EV = 100000
NW = 32
RPW = ROWS // NW
IPW = RPW * L
CHUNK = 128
NCHUNK = IPW // CHUNK
NCHUNK_PAD = 104
SUP = 10
NSUP = NCHUNK // SUP
PBLK = 2048


def _proj_body(tab_ref, m_ref, *o_refs):
    m = m_ref[...].astype(jnp.bfloat16)
    out = jax.lax.dot_general(m, tab_ref[...].astype(jnp.bfloat16),
                              (((1,), (1,)), ((), ())),
                              preferred_element_type=jnp.float32)

    def bf_hi(x):
        r = x.astype(jnp.bfloat16).astype(jnp.float32)
        return jax.lax.bitcast_convert_type(r, jnp.uint32) & jnp.uint32(
            0xFFFF0000)

    for q, o_ref in enumerate(o_refs):
        packed = bf_hi(out[2 * q]) | (bf_hi(out[2 * q + 1]) >> 16)
        o_ref[...] = jax.lax.bitcast_convert_type(packed, jnp.int32)


def _project(table128, m, ncols):
    v128, _ = table128.shape
    return pl.pallas_call(
        _proj_body,
        grid=(pl.cdiv(v128, PBLK),),
        in_specs=[
            pl.BlockSpec((PBLK, 128), lambda i: (i, 0)),
            pl.BlockSpec((128, 128), lambda i: (0, 0)),
        ],
        out_specs=[pl.BlockSpec((PBLK,), lambda i: (i,))] * ncols,
        out_shape=[jax.ShapeDtypeStruct((v128,), jnp.int32)] * ncols,
    )(table128, m)


def _pool_body(wcat_hbm, ecat_hbm, wpi_hbm, epi_hbm,
               gidx_hbm, z_hbm, ow0_hbm, ow1_hbm, oe0_hbm, oe1_hbm,
               idx_v, g_v, buf0, buf1, abuf, bbuf, wtab_s, etab_s,
               aw0_s, aw1_s, ae0_s, ae1_s, sem0, sem1, ssem):
    c = lax.axis_index("c")
    s = lax.axis_index("s")
    w = c * 16 + s

    pltpu.sync_copy(gidx_hbm.at[s], g_v)

    @pl.when(s < 8)
    def _():
        pltpu.sync_copy(wcat_hbm.at[pl.ds(s * (WV // 8), WV // 8)],
                        wtab_s.at[pl.ds(s * (WV // 8), WV // 8)])

    @pl.when(s < 4)
    def _():
        pltpu.sync_copy(ecat_hbm.at[pl.ds(s * (EV // 4), EV // 4)],
                        etab_s.at[pl.ds(s * (EV // 4), EV // 4)])

    for acc in (aw0_s, aw1_s, ae0_s, ae1_s):
        pltpu.sync_copy(z_hbm, acc.at[pl.ds(s * RPW, RPW)])
    plsc.subcore_barrier()

    def phase(col_hbm, idxsrc_hbm, acc0_s, acc1_s):
        pltpu.sync_copy(idxsrc_hbm.at[w], idx_v)
        sc = SUP * CHUNK

        def fire(i, buf, sem):
            src = col_hbm.at[idx_v.at[pl.ds(i * sc, sc)]]
            pltpu.async_copy(src, buf, sem)

        def drain(i, buf, sem):
            src = col_hbm.at[idx_v.at[pl.ds(i * sc, sc)]]
            pltpu.make_async_copy(src, buf, sem).wait()

        def unpack_scat(i, buf):
            for r in range(sc // 16):
                v = buf[pl.ds(r * 16, 16)]
                abuf[pl.ds(r * 16, 16)] = plsc.bitcast(
                    v & jnp.int32(-65536), jnp.float32)
                bbuf[pl.ds(r * 16, 16)] = plsc.bitcast(
                    jax.lax.shift_left(v, 16), jnp.float32)
            g = g_v.at[pl.ds(i * sc, sc)]
            d0 = pltpu.async_copy(abuf, acc0_s.at[g], ssem, add=True)
            d1 = pltpu.async_copy(bbuf, acc1_s.at[g], ssem, add=True)
            d0.wait()
            d1.wait()

        fire(0, buf0, sem0)

        def step(j, carry):
            i = 2 * j
            drain(i, buf0, sem0)
            fire(i + 1, buf1, sem1)
            unpack_scat(i, buf0)
            drain(i + 1, buf1, sem1)

            @pl.when(i + 2 < NSUP)
            def _():
                fire(i + 2, buf0, sem0)

            unpack_scat(i + 1, buf1)
            return carry

        lax.fori_loop(0, NSUP // 2, step, 0, unroll=False)

    phase(wtab_s, wpi_hbm, aw0_s, aw1_s)
    phase(etab_s, epi_hbm, ae0_s, ae1_s)

    for acc, out in ((aw0_s, ow0_hbm), (aw1_s, ow1_hbm),
                     (ae0_s, oe0_hbm), (ae1_s, oe1_hbm)):
        pltpu.sync_copy(acc.at[pl.ds(s * RPW, RPW)],
                        out.at[pl.ds(w * RPW, RPW)])


def _scorer_body(params_ref, er_ref, ecw_ref, el_ref, ece_ref, out_ref):
    er_raw = er_ref[...]
    el_raw = el_ref[...]
    ec_raw = ecw_ref[...] + ece_ref[...]
    er_b, el_b, ec_b, cl_b = (params_ref[0], params_ref[1], params_ref[2],
                              params_ref[3])
    cl0, cl1, cl2 = params_ref[4], params_ref[5], params_ref[6]
    er = jax.nn.relu(er_raw + er_b - 0.5) + 0.5
    el = jax.nn.relu(el_raw + el_b - 0.5) + 0.5
    ec = jax.nn.sigmoid(ec_raw + ec_b)
    out_ref[...] = jax.nn.sigmoid(er * cl0 + el * cl1 + ec * cl2 + cl_b)


def _projmat(w0, w1, d, packs):
    m = jnp.zeros((128, 128), jnp.float32)
    for q in range(packs):
        m = m.at[2 * q, q * d:(q + 1) * d].set(w0)
        m = m.at[2 * q + 1, q * d:(q + 1) * d].set(w1)
    return m


def kernel(lctx_words, rctx_words, lctx_entities, rctx_entities,
           word_table, entity_table, er_w, er_b, el_w, el_b,
           ec_w, ec_b, cl_w, cl_b):
    inv_l = 1.0 / L
    wcols = _project(word_table.reshape(-1, 128),
                     _projmat(er_w[:, 0] * inv_l, ec_w[:WD, 0] * inv_l,
                              WD, 128 // WD), 128 // WD)
    ecols = _project(entity_table.reshape(-1, 128),
                     _projmat(el_w[:, 0] * inv_l, ec_w[WD:, 0] * inv_l,
                              ED, 128 // ED), 128 // ED)
    wcat = jnp.concatenate(wcols)
    ecat = jnp.concatenate(ecols)

    widx = jnp.concatenate([rctx_words, lctx_words], axis=0).reshape(
        NW, IPW)
    eidx = jnp.concatenate([rctx_entities, lctx_entities], axis=0).reshape(
        NW, IPW)
    wpi = (widx & 3) * (WV // 4) + (widx >> 2)
    epi = (eidx & 1) * (EV // 2) + (eidx >> 1)

    def perm(a):
        return a.reshape(NW, CHUNK, NCHUNK).swapaxes(1, 2).reshape(NW, IPW)

    wpi, epi = perm(wpi), perm(epi)

    seg = (jnp.arange(IPW, dtype=jnp.int32) // L).reshape(1, IPW)
    seg = perm(jnp.broadcast_to(seg, (NW, IPW)))[:1]
    gidx = seg + (RPW * jnp.arange(16, dtype=jnp.int32))[:, None]
    z = jnp.zeros((RPW,), jnp.float32)

    mesh = plsc.VectorSubcoreMesh(core_axis_name="c", subcore_axis_name="s")
    pool = pl.kernel(
        _pool_body,
        out_type=tuple(jax.ShapeDtypeStruct((ROWS,), jnp.float32)
                       for _ in range(4)),
        mesh=mesh,
        compiler_params=pltpu.CompilerParams(use_tc_tiling_on_sc=False,
                                             needs_layout_passes=False),
        scratch_types=[
            pltpu.VMEM((IPW,), jnp.int32),
            pltpu.VMEM((IPW,), jnp.int32),
            pltpu.VMEM((SUP * CHUNK,), jnp.int32),
            pltpu.VMEM((SUP * CHUNK,), jnp.int32),
            pltpu.VMEM((SUP * CHUNK,), jnp.float32),
            pltpu.VMEM((SUP * CHUNK,), jnp.float32),
            pltpu.VMEM_SHARED((WV,), jnp.int32),
            pltpu.VMEM_SHARED((EV,), jnp.int32),
            pltpu.VMEM_SHARED((16 * RPW,), jnp.float32),
            pltpu.VMEM_SHARED((16 * RPW,), jnp.float32),
            pltpu.VMEM_SHARED((16 * RPW,), jnp.float32),
            pltpu.VMEM_SHARED((16 * RPW,), jnp.float32),
            pltpu.SemaphoreType.DMA,
            pltpu.SemaphoreType.DMA,
            pltpu.SemaphoreType.DMA,
        ],
    )
    ow0, ow1, oe0, oe1 = pool(wcat, ecat, wpi, epi, gidx, z)

    params = jnp.concatenate([er_b, el_b, ec_b, cl_b, cl_w[:, 0]])
    final = pl.pallas_call(
        _scorer_body,
        out_shape=jax.ShapeDtypeStruct((ROWS,), jnp.float32),
        in_specs=[pl.BlockSpec(memory_space=pltpu.SMEM)] +
                 [pl.BlockSpec(memory_space=pltpu.VMEM)] * 4,
    )(params, ow0, ow1, oe0, oe1)
    return final.reshape(ROWS, 1)
